# Initial kernel scaffold; baseline (speedup 1.0000x reference)
import jax
import jax.numpy as jnp
from jax import lax
from jax.experimental import pallas as pl
from jax.experimental.pallas import tpu as pltpu
from jax.experimental.pallas import tpu_sc as plsc

B, L, V, D = 4096, 20, 100000, 4
LANES = 16
NC, NS = 2, 16
ROWS_PER_W = B // NS
RCHUNKS = ROWS_PER_W // LANES


def _project_body(t1_ref, t2_ref, m_ref, o1_ref, o2_ref):
    m = m_ref[...]
    o1_ref[...] = jnp.dot(t1_ref[...], m, preferred_element_type=jnp.float32)
    o2_ref[...] = jnp.dot(t2_ref[...], m, preferred_element_type=jnp.float32)


def _pool_body(s1, s2, idx1, idx2, bvec, out, table_v, idx_v, out_v, b_v):
    c = lax.axis_index("c")
    s = lax.axis_index("s")
    base = s * ROWS_PER_W

    @pl.when(c == 0)
    def _():
        pltpu.sync_copy(s1, table_v)
        pltpu.sync_copy(idx1.at[pl.ds(base, ROWS_PER_W), :], idx_v)

    @pl.when(c == 1)
    def _():
        pltpu.sync_copy(s2, table_v)
        pltpu.sync_copy(idx2.at[pl.ds(base, ROWS_PER_W), :], idx_v)

    pltpu.sync_copy(bvec, b_v)
    bias = b_v[...]
    iota = lax.iota(jnp.int32, LANES)

    def rc_body(rc, carry):
        rows = rc * LANES + iota
        acc = bias
        for j in range(L):
            jv = jnp.full((LANES,), j, jnp.int32)
            tidx = plsc.load_gather(idx_v, [rows, jv])
            acc = acc + plsc.load_gather(table_v, [tidx])
        plsc.store_scatter(out_v, [rows], acc)
        return carry

    lax.fori_loop(0, RCHUNKS, rc_body, 0)
    pltpu.sync_copy(out_v, out.at[pl.ds(c * B + base, ROWS_PER_W)])


def kernel(indices_f1, indices_f2, table_f1, table_f2, W, b):
    t1r = table_f1.reshape(V // 32, 128)
    t2r = table_f2.reshape(V // 32, 128)
    key = (jnp.arange(128, dtype=jnp.int32) // D)[:, None] == jnp.arange(
        32, dtype=jnp.int32)[None, :]
    m = key.astype(jnp.float32) * jnp.tile(W[0], 32)[:, None]

    s1p, s2p = pl.pallas_call(
        _project_body,
        out_shape=[jax.ShapeDtypeStruct((V // 32, 32), jnp.float32)] * 2,
    )(t1r, t2r, m)
    s1 = s1p.reshape(V)
    s2 = s2p.reshape(V)
    bvec = jnp.tile(b, LANES)

    mesh = plsc.VectorSubcoreMesh(core_axis_name="c", subcore_axis_name="s")
    pool = pl.kernel(
        _pool_body,
        mesh=mesh,
        out_type=jax.ShapeDtypeStruct((2 * B,), jnp.float32),
        scratch_types=[
            pltpu.VMEM((V,), jnp.float32),
            pltpu.VMEM((ROWS_PER_W, L), jnp.int32),
            pltpu.VMEM((ROWS_PER_W,), jnp.float32),
            pltpu.VMEM((LANES,), jnp.float32),
        ],
    )
    out = pool(s1, s2, indices_f1, indices_f2, bvec)
    return out.reshape(2 * B, 1)


# speedup vs baseline: 1.1042x; 1.1042x over previous
import jax
import jax.numpy as jnp
from jax import lax
from jax.experimental import pallas as pl
from jax.experimental.pallas import tpu as pltpu
from jax.experimental.pallas import tpu_sc as plsc

B, L, V, D = 4096, 20, 100000, 4
LANES = 16
NC, NS = 2, 16
ROWS_PER_W = B // NS
RCHUNKS = ROWS_PER_W // LANES


def _project_body(t1_ref, t2_ref, m_ref, o1_ref, o2_ref):
    m = m_ref[...]
    o1_ref[...] = jnp.dot(t1_ref[...], m, preferred_element_type=jnp.float32)
    o2_ref[...] = jnp.dot(t2_ref[...], m, preferred_element_type=jnp.float32)


def _pool_body(s1, s2, idx1, idx2, bvec, out, table_v, idx_v, out_v, b_v):
    c = lax.axis_index("c")
    s = lax.axis_index("s")
    base = s * ROWS_PER_W

    @pl.when(c == 0)
    def _():
        pltpu.sync_copy(s1, table_v)
        pltpu.sync_copy(idx1.at[pl.ds(base * L, ROWS_PER_W * L)], idx_v)

    @pl.when(c == 1)
    def _():
        pltpu.sync_copy(s2, table_v)
        pltpu.sync_copy(idx2.at[pl.ds(base * L, ROWS_PER_W * L)], idx_v)

    pltpu.sync_copy(bvec, b_v)
    bias = b_v[...]
    iota = lax.iota(jnp.int32, LANES)

    def rc_body(rc, carry):
        rows = rc * LANES + iota
        flat0 = rows * L
        acc = bias
        for j in range(L):
            tidx = plsc.load_gather(idx_v, [flat0 + j])
            acc = acc + plsc.load_gather(table_v, [tidx])
        plsc.store_scatter(out_v, [rows], acc)
        return carry

    lax.fori_loop(0, RCHUNKS, rc_body, 0)
    pltpu.sync_copy(out_v, out.at[pl.ds(c * B + base, ROWS_PER_W)])


def kernel(indices_f1, indices_f2, table_f1, table_f2, W, b):
    t1r = table_f1.reshape(V // 32, 128)
    t2r = table_f2.reshape(V // 32, 128)
    key = (jnp.arange(128, dtype=jnp.int32) // D)[:, None] == jnp.arange(
        32, dtype=jnp.int32)[None, :]
    m = key.astype(jnp.float32) * jnp.tile(W[0], 32)[:, None]

    s1p, s2p = pl.pallas_call(
        _project_body,
        out_shape=[jax.ShapeDtypeStruct((V // 32, 32), jnp.float32)] * 2,
    )(t1r, t2r, m)
    s1 = s1p.reshape(V)
    s2 = s2p.reshape(V)
    bvec = jnp.tile(b, LANES)

    mesh = plsc.VectorSubcoreMesh(core_axis_name="c", subcore_axis_name="s")
    pool = pl.kernel(
        _pool_body,
        mesh=mesh,
        out_type=jax.ShapeDtypeStruct((2 * B,), jnp.float32),
        compiler_params=pltpu.CompilerParams(needs_layout_passes=False),
        scratch_types=[
            pltpu.VMEM((V,), jnp.float32),
            pltpu.VMEM((ROWS_PER_W * L,), jnp.int32),
            pltpu.VMEM((ROWS_PER_W,), jnp.float32),
            pltpu.VMEM((LANES,), jnp.float32),
        ],
    )
    out = pool(s1, s2, indices_f1.reshape(-1), indices_f2.reshape(-1), bvec)
    return out.reshape(2 * B, 1)


# speedup vs baseline: 4.4902x; 4.0663x over previous
import jax
import jax.numpy as jnp
from jax import lax
from jax.experimental import pallas as pl
from jax.experimental.pallas import tpu as pltpu
from jax.experimental.pallas import tpu_sc as plsc

B, L, V, D = 4096, 20, 100000, 4
LANES = 16
NC, NS = 2, 16
ROWS_PER_W = B // NS
RCHUNKS = ROWS_PER_W // LANES


def _project_body(t1_ref, t2_ref, w_ref, o1_ref, o2_ref):
    w = w_ref[...]
    o1_ref[...] = jnp.sum(t1_ref[...] * w, axis=0)
    o2_ref[...] = jnp.sum(t2_ref[...] * w, axis=0)


def _pool_body(s1, s2, idx1, idx2, bvec, out, table_v, idx_v, out_v, b_v):
    c = lax.axis_index("c")
    s = lax.axis_index("s")
    base = s * ROWS_PER_W

    @pl.when(c == 0)
    def _():
        pltpu.sync_copy(s1, table_v)
        pltpu.sync_copy(idx1.at[pl.ds(base * L, ROWS_PER_W * L)], idx_v)

    @pl.when(c == 1)
    def _():
        pltpu.sync_copy(s2, table_v)
        pltpu.sync_copy(idx2.at[pl.ds(base * L, ROWS_PER_W * L)], idx_v)

    pltpu.sync_copy(bvec, b_v)
    bias = b_v[...]
    iota = lax.iota(jnp.int32, LANES)

    def rc_body(rc, carry):
        rows = rc * LANES + iota
        flat0 = rows * L
        acc = bias
        for j in range(L):
            tidx = plsc.load_gather(idx_v, [flat0 + j])
            acc = acc + plsc.load_gather(table_v, [tidx])
        plsc.store_scatter(out_v, [rows], acc)
        return carry

    lax.fori_loop(0, RCHUNKS, rc_body, 0)
    pltpu.sync_copy(out_v, out.at[pl.ds(c * B + base, ROWS_PER_W)])


def kernel(indices_f1, indices_f2, table_f1, table_f2, W, b):
    s1, s2 = pl.pallas_call(
        _project_body,
        out_shape=[jax.ShapeDtypeStruct((V,), jnp.float32)] * 2,
    )(table_f1.T, table_f2.T, W.T)
    bvec = jnp.tile(b, LANES)

    mesh = plsc.VectorSubcoreMesh(core_axis_name="c", subcore_axis_name="s")
    pool = pl.kernel(
        _pool_body,
        mesh=mesh,
        out_type=jax.ShapeDtypeStruct((2 * B,), jnp.float32),
        compiler_params=pltpu.CompilerParams(needs_layout_passes=False),
        scratch_types=[
            pltpu.VMEM((V,), jnp.float32),
            pltpu.VMEM((ROWS_PER_W * L,), jnp.int32),
            pltpu.VMEM((ROWS_PER_W,), jnp.float32),
            pltpu.VMEM((LANES,), jnp.float32),
        ],
    )
    out = pool(s1, s2, indices_f1.reshape(-1), indices_f2.reshape(-1), bvec)
    return out.reshape(2 * B, 1)
